# Initial kernel scaffold; baseline (speedup 1.0000x reference)
import jax
import jax.numpy as jnp
from jax.experimental import pallas as pl


def kernel(num_tokens, unit_tokens, num_table, unit_table):
    raise NotImplementedError("write your pallas kernel here")


# speedup vs baseline: 4.3065x; 4.3065x over previous
import functools

import jax
import jax.numpy as jnp
from jax import lax
from jax.experimental import pallas as pl
from jax.experimental.pallas import tpu as pltpu
from jax.experimental.pallas import tpu_sc as plsc

EMBED = 64
B = 4096 * 50


@functools.lru_cache(maxsize=None)
def _build(chunk: int):
    info = plsc.get_sparse_core_info()
    nc, ns = info.num_cores, info.num_subcores
    nw = nc * ns
    b_per_w = B // nw
    n_chunks = b_per_w // chunk

    mesh = plsc.VectorSubcoreMesh(core_axis_name="c", subcore_axis_name="s")

    @functools.partial(
        pl.kernel,
        mesh=mesh,
        compiler_params=pltpu.CompilerParams(use_tc_tiling_on_sc=False),
        out_type=(
            jax.ShapeDtypeStruct((B, EMBED), jnp.float32),
            jax.ShapeDtypeStruct((B, EMBED), jnp.float32),
        ),
        scratch_types=[
            pltpu.VMEM((n_chunks, chunk), jnp.int32),
            pltpu.VMEM((chunk, EMBED), jnp.float32),
            pltpu.SemaphoreType.DMA,
        ],
    )
    def gather_kernel(num_idx, unit_idx, num_tab, unit_tab,
                      num_out, unit_out, idx_v, rows_v, sem):
        wid = lax.axis_index("s") * nc + lax.axis_index("c")
        base = wid * b_per_w
        for idx_hbm, tab_hbm, out_hbm in (
            (num_idx, num_tab, num_out),
            (unit_idx, unit_tab, unit_out),
        ):
            pltpu.sync_copy(idx_hbm.at[wid], idx_v)

            def body(c, _, tab_hbm=tab_hbm, out_hbm=out_hbm):
                pltpu.async_copy(tab_hbm.at[idx_v.at[c]], rows_v, sem).wait()
                pltpu.sync_copy(rows_v,
                                out_hbm.at[pl.ds(base + c * chunk, chunk)])
                return 0

            lax.fori_loop(0, n_chunks, body, 0)

    return gather_kernel, nw, n_chunks


def kernel(num_tokens, unit_tokens, num_table, unit_table):
    chunk = 128
    fn, nw, n_chunks = _build(chunk)
    shape = num_tokens.shape
    num_idx = num_tokens.reshape(nw, n_chunks, chunk).astype(jnp.int32)
    unit_idx = unit_tokens.reshape(nw, n_chunks, chunk).astype(jnp.int32)
    num_out, unit_out = fn(num_idx, unit_idx, num_table, unit_table)
    return (num_out.reshape(*shape, EMBED), unit_out.reshape(*shape, EMBED))


# speedup vs baseline: 4.9851x; 1.1576x over previous
import functools

import jax
import jax.numpy as jnp
from jax import lax
from jax.experimental import pallas as pl
from jax.experimental.pallas import tpu as pltpu
from jax.experimental.pallas import tpu_sc as plsc

EMBED = 64
B = 4096 * 50


@functools.lru_cache(maxsize=None)
def _build(chunk: int):
    info = plsc.get_sparse_core_info()
    nc, ns = info.num_cores, info.num_subcores
    nw = nc * ns
    b_per_w = B // nw
    n_chunks = b_per_w // chunk

    mesh = plsc.VectorSubcoreMesh(core_axis_name="c", subcore_axis_name="s")

    @functools.partial(
        pl.kernel,
        mesh=mesh,
        compiler_params=pltpu.CompilerParams(use_tc_tiling_on_sc=False),
        out_type=(
            jax.ShapeDtypeStruct((B, EMBED), jnp.float32),
            jax.ShapeDtypeStruct((B, EMBED), jnp.float32),
        ),
        scratch_types=[
            pltpu.VMEM((n_chunks, chunk), jnp.int32),
            pltpu.VMEM((chunk, EMBED), jnp.float32),
            pltpu.SemaphoreType.DMA,
        ],
    )
    def gather_kernel(num_idx, unit_idx, num_tab, unit_tab,
                      num_out, unit_out, idx_v, rows_v, sem):
        wid = lax.axis_index("s") * nc + lax.axis_index("c")
        base = wid * b_per_w
        for idx_hbm, tab_hbm, out_hbm in (
            (num_idx, num_tab, num_out),
            (unit_idx, unit_tab, unit_out),
        ):
            pltpu.sync_copy(idx_hbm.at[wid], idx_v)

            def body(c, _, tab_hbm=tab_hbm, out_hbm=out_hbm):
                pltpu.async_copy(tab_hbm.at[idx_v.at[c]], rows_v, sem).wait()
                pltpu.sync_copy(rows_v,
                                out_hbm.at[pl.ds(base + c * chunk, chunk)])
                return 0

            lax.fori_loop(0, n_chunks, body, 0)

    return gather_kernel, nw, n_chunks


def kernel(num_tokens, unit_tokens, num_table, unit_table):
    chunk = 1600
    fn, nw, n_chunks = _build(chunk)
    shape = num_tokens.shape
    num_idx = num_tokens.reshape(nw, n_chunks, chunk).astype(jnp.int32)
    unit_idx = unit_tokens.reshape(nw, n_chunks, chunk).astype(jnp.int32)
    num_out, unit_out = fn(num_idx, unit_idx, num_table, unit_table)
    return (num_out.reshape(*shape, EMBED), unit_out.reshape(*shape, EMBED))
